# baseline (speedup 1.0000x reference)
import functools

import jax
import jax.numpy as jnp
from jax import lax
from jax.experimental import pallas as pl
from jax.experimental.pallas import tpu as pltpu
from jax.experimental.pallas import tpu_sc as plsc

_BATCH = 16384
_NF = 26
_NC = 2
_NS = 16
_NW = _NC * _NS
_RPW = _BATCH // _NW
_IPW = _RPW * _NF
_CHUNKS = _RPW // 16


_GROUPS = ((0, 7), (7, 13), (13, 20), (20, 26))


def _sc_body(xt_hbm, table_hbm, bias_hbm, out_hbm, idx_v, val_v, out_v, bias_v,
             sem, gsem0, gsem1, gsem2, gsem3, bsem):
    wid = lax.axis_index("s") * _NC + lax.axis_index("c")
    base = wid * _RPW

    pltpu.async_copy(bias_hbm, bias_v.at[pl.ds(0, 1)], bsem)

    stage = [
        pltpu.async_copy(
            xt_hbm.at[f, pl.ds(base, _RPW)],
            idx_v.at[pl.ds(f * _RPW, _RPW)],
            sem,
        )
        for f in range(_NF)
    ]
    tbl = table_hbm.at[0]
    gsems = (gsem0, gsem1, gsem2, gsem3)

    gathers = []
    for (f0, f1), gs in zip(_GROUPS, gsems):
        for d in stage[f0:f1]:
            d.wait()
        lo, n = f0 * _RPW, (f1 - f0) * _RPW
        gathers.append(
            pltpu.async_copy(
                tbl.at[idx_v.at[pl.ds(lo, n)]], val_v.at[pl.ds(lo, n)], gs
            )
        )

    pltpu.make_async_copy(bias_hbm, bias_v.at[pl.ds(0, 1)], bsem).wait()
    b = bias_v[...][0]

    for gi, ((f0, f1), g) in enumerate(zip(_GROUPS, gathers)):
        g.wait()
        first, last = gi == 0, gi == len(_GROUPS) - 1

        def grp_body(c, carry, f0=f0, f1=f1, first=first, last=last):
            acc = b if first else out_v[pl.ds(c * 16, 16)]
            for f in range(f0, f1):
                acc = acc + val_v[pl.ds(f * _RPW + c * 16, 16)]
            out_v[pl.ds(c * 16, 16)] = acc
            return carry

        lax.fori_loop(0, _CHUNKS, grp_body, 0)

    pltpu.sync_copy(out_v, out_hbm.at[pl.ds(base, _RPW)])


_sc_call = pl.kernel(
    _sc_body,
    out_type=jax.ShapeDtypeStruct((_BATCH,), jnp.float32),
    mesh=plsc.VectorSubcoreMesh(core_axis_name="c", subcore_axis_name="s"),
    scratch_types=[
        pltpu.VMEM((_IPW,), jnp.int32),
        pltpu.VMEM((_IPW,), jnp.float32),
        pltpu.VMEM((_RPW,), jnp.float32),
        pltpu.VMEM((16,), jnp.float32),
        pltpu.SemaphoreType.DMA,
        pltpu.SemaphoreType.DMA,
        pltpu.SemaphoreType.DMA,
        pltpu.SemaphoreType.DMA,
        pltpu.SemaphoreType.DMA,
        pltpu.SemaphoreType.DMA,
    ],
    compiler_params=pltpu.CompilerParams(needs_layout_passes=False),
)


@jax.jit
def kernel(x, table, bias):
    out = _sc_call(x.T, table.T, bias)
    return out.reshape(-1, 1)


# speedup vs baseline: 1.0004x; 1.0004x over previous
import functools

import jax
import jax.numpy as jnp
from jax import lax
from jax.experimental import pallas as pl
from jax.experimental.pallas import tpu as pltpu
from jax.experimental.pallas import tpu_sc as plsc

_BATCH = 16384
_NF = 26
_NC = 2
_NS = 16
_NW = _NC * _NS
_RPW = _BATCH // _NW
_IPW = _RPW * _NF
_CHUNKS = _RPW // 16


_GROUPS = ((0, 7), (7, 13), (13, 20), (20, 26))


def _sc_body(xt_hbm, table_hbm, bias_hbm, out_hbm, idx_v, val_v, out_v, bias_v,
             sem, gsem0, gsem1, gsem2, gsem3, bsem):
    wid = lax.axis_index("s") * _NC + lax.axis_index("c")
    base = wid * _RPW

    pltpu.async_copy(bias_hbm, bias_v.at[pl.ds(0, 1)], bsem)

    stage = [
        pltpu.async_copy(
            xt_hbm.at[f, pl.ds(base, _RPW)],
            idx_v.at[pl.ds(f * _RPW, _RPW)],
            sem,
        )
        for f in range(_NF)
    ]
    tbl = table_hbm.at[0]
    gsems = (gsem0, gsem1, gsem2, gsem3)

    gathers = []
    for (f0, f1), gs in zip(_GROUPS, gsems):
        for d in stage[f0:f1]:
            d.wait()
        lo, n = f0 * _RPW, (f1 - f0) * _RPW
        gathers.append(
            pltpu.async_copy(
                tbl.at[idx_v.at[pl.ds(lo, n)]], val_v.at[pl.ds(lo, n)], gs
            )
        )

    pltpu.make_async_copy(bias_hbm, bias_v.at[pl.ds(0, 1)], bsem).wait()
    b = bias_v[...][0]

    for gi, ((f0, f1), g) in enumerate(zip(_GROUPS, gathers)):
        g.wait()
        first, last = gi == 0, gi == len(_GROUPS) - 1

        def grp_body(c, carry, f0=f0, f1=f1, first=first, last=last):
            acc = b if first else out_v[pl.ds(c * 16, 16)]
            for f in range(f0, f1):
                acc = acc + val_v[pl.ds(f * _RPW + c * 16, 16)]
            out_v[pl.ds(c * 16, 16)] = acc
            return carry

        lax.fori_loop(0, _CHUNKS, grp_body, 0)

    pltpu.sync_copy(out_v, out_hbm.at[pl.ds(base, _RPW)])


_sc_call = pl.kernel(
    _sc_body,
    out_type=jax.ShapeDtypeStruct((_BATCH,), jnp.float32),
    mesh=plsc.VectorSubcoreMesh(core_axis_name="c", subcore_axis_name="s"),
    scratch_types=[
        pltpu.VMEM((_IPW,), jnp.int32),
        pltpu.VMEM((_IPW,), jnp.float32),
        pltpu.VMEM((_RPW,), jnp.float32),
        pltpu.VMEM((16,), jnp.float32),
        pltpu.SemaphoreType.DMA,
        pltpu.SemaphoreType.DMA,
        pltpu.SemaphoreType.DMA,
        pltpu.SemaphoreType.DMA,
        pltpu.SemaphoreType.DMA,
        pltpu.SemaphoreType.DMA,
    ],
    compiler_params=pltpu.CompilerParams(
        needs_layout_passes=False,
        disable_bounds_checks=True,
        skip_device_barrier=True,
    ),
)


@jax.jit
def kernel(x, table, bias):
    out = _sc_call(x.T, table.T, bias)
    return out.reshape(-1, 1)


# speedup vs baseline: 1.0012x; 1.0008x over previous
import functools

import jax
import jax.numpy as jnp
from jax import lax
from jax.experimental import pallas as pl
from jax.experimental.pallas import tpu as pltpu
from jax.experimental.pallas import tpu_sc as plsc

_BATCH = 16384
_NF = 26
_NC = 2
_NS = 16
_NW = _NC * _NS
_RPW = _BATCH // _NW
_IPW = _RPW * _NF
_CHUNKS = _RPW // 16


_GROUPS = ((0, 7), (7, 13), (13, 20), (20, 26))


def _sc_body(xt_hbm, table_hbm, bias_hbm, out_hbm, idx_v, val_v, out_v, bias_v,
             sem, gsem0, gsem1, gsem2, gsem3, bsem):
    wid = lax.axis_index("s") * _NC + lax.axis_index("c")
    base = wid * _RPW

    pltpu.async_copy(bias_hbm, bias_v.at[pl.ds(0, 1)], bsem)

    stage = [
        pltpu.async_copy(
            xt_hbm.at[f, pl.ds(base, _RPW)],
            idx_v.at[pl.ds(f * _RPW, _RPW)],
            sem,
        )
        for f in range(_NF)
    ]
    tbl = table_hbm.at[0]
    gsems = (gsem0, gsem1, gsem2, gsem3)

    gathers = []
    for (f0, f1), gs in zip(_GROUPS, gsems):
        for d in stage[f0:f1]:
            d.wait()
        lo, n = f0 * _RPW, (f1 - f0) * _RPW
        gathers.append(
            pltpu.async_copy(
                tbl.at[idx_v.at[pl.ds(lo, n)]], val_v.at[pl.ds(lo, n)], gs
            )
        )

    pltpu.make_async_copy(bias_hbm, bias_v.at[pl.ds(0, 1)], bsem).wait()
    b = bias_v[...][0]

    for gi, ((f0, f1), g) in enumerate(zip(_GROUPS, gathers)):
        g.wait()
        first, last = gi == 0, gi == len(_GROUPS) - 1

        def grp_body(c, carry, f0=f0, f1=f1, first=first, last=last):
            acc = b if first else out_v[pl.ds(c * 16, 16)]
            for f in range(f0, f1):
                acc = acc + val_v[pl.ds(f * _RPW + c * 16, 16)]
            out_v[pl.ds(c * 16, 16)] = acc
            return carry

        lax.fori_loop(0, _CHUNKS, grp_body, 0)

    pltpu.sync_copy(out_v, out_hbm.at[pl.ds(base, _RPW)])


_sc_call = pl.kernel(
    _sc_body,
    out_type=jax.ShapeDtypeStruct((_BATCH,), jnp.float32),
    mesh=plsc.VectorSubcoreMesh(core_axis_name="c", subcore_axis_name="s"),
    scratch_types=[
        pltpu.VMEM((_IPW,), jnp.int32),
        pltpu.VMEM((_IPW,), jnp.float32),
        pltpu.VMEM((_RPW,), jnp.float32),
        pltpu.VMEM((16,), jnp.float32),
        pltpu.SemaphoreType.DMA,
        pltpu.SemaphoreType.DMA,
        pltpu.SemaphoreType.DMA,
        pltpu.SemaphoreType.DMA,
        pltpu.SemaphoreType.DMA,
        pltpu.SemaphoreType.DMA,
    ],
    compiler_params=pltpu.CompilerParams(needs_layout_passes=False),
)


@jax.jit
def kernel(x, table, bias):
    out = _sc_call(x.T, table.T, bias)
    return out.reshape(-1, 1)
